# baseline (speedup 1.0000x reference)
import functools

import jax
import jax.numpy as jnp
from jax.experimental import pallas as pl
from jax.experimental.pallas import tpu as pltpu


def _mlp_kernel(x_ref, p0_ref, p1_ref, p2_ref, o_ref):
    x = x_ref[0]
    tn = x.shape[-1]
    one = jnp.ones((1, tn), jnp.float32)
    h = jnp.dot(p0_ref[...], jnp.concatenate([x, one], axis=0),
                preferred_element_type=jnp.float32)
    h = jnp.maximum(h, 0.0)
    h = jnp.dot(p1_ref[...], jnp.concatenate([h, one], axis=0),
                preferred_element_type=jnp.float32)
    h = jnp.maximum(h, 0.0)
    o_ref[0] = jnp.dot(p2_ref[...], jnp.concatenate([h, one], axis=0),
                       preferred_element_type=jnp.float32)


def _sample_agg_kernel(score_ref, coords_ref, w_ref, o_ref, *,
                       depth, height, width, nbr, dchunk):
    tp = o_ref.shape[-1]

    xs = jax.lax.broadcasted_iota(jnp.int32, (width, tp), 0).astype(jnp.float32)
    ys = jax.lax.broadcasted_iota(jnp.int32, (height, tp), 0).astype(jnp.float32)

    axs = []
    ays = []
    for n in range(nbr):
        gx = coords_ref[0, n:n + 1, :]
        gy = coords_ref[0, nbr + n:nbr + n + 1, :]

        ix = jnp.clip((gx + 1.0) * (0.5 * width) - 0.5, 0.0, width - 1.0)
        iy = jnp.clip((gy + 1.0) * (0.5 * height) - 0.5, 0.0, height - 1.0)
        x0 = jnp.floor(ix)
        y0 = jnp.floor(iy)
        wx = ix - x0
        wy = iy - y0
        x1 = jnp.minimum(x0 + 1.0, width - 1.0)
        y1 = jnp.minimum(y0 + 1.0, height - 1.0)

        axs.append(jnp.where(xs == x0, 1.0 - wx, 0.0)
                   + jnp.where(xs == x1, wx, 0.0))
        ays.append(jnp.where(ys == y0, 1.0 - wy, 0.0)
                   + jnp.where(ys == y1, wy, 0.0))

    nchunks = depth // dchunk
    accs = [None] * nchunks
    for n in range(nbr):
        ay = ays[n][None, :, :]
        for c in range(nchunks):
            rows = slice(c * dchunk * height, (c + 1) * dchunk * height)
            t = jnp.dot(score_ref[0, rows, :], axs[n],
                        preferred_element_type=jnp.float32)
            r = jnp.sum(t.reshape(dchunk, height, tp) * ay, axis=1)
            r = r * w_ref[0, c * dchunk:(c + 1) * dchunk, n, :]
            accs[c] = r if accs[c] is None else accs[c] + r
    o_ref[0] = jnp.concatenate(accs, axis=0)


def _lane_tile(n, cap):
    t = cap
    while t > 128 and n % t:
        t -= 128
    return t


def kernel(x1, grid, weight, p0, p1, p2):
    x1 = x1.astype(jnp.float32)
    grid = grid.astype(jnp.float32)
    weight = weight.astype(jnp.float32)
    p0 = p0.astype(jnp.float32)
    p1 = p1.astype(jnp.float32)
    p2 = p2.astype(jnp.float32)

    B, G, D, H, W = x1.shape
    nbr = weight.shape[2]
    HW = H * W
    N = D * HW
    vmem = dict(vmem_limit_bytes=100 * 1024 * 1024)

    TN = _lane_tile(N, 16384)
    score = pl.pallas_call(
        _mlp_kernel,
        out_shape=jax.ShapeDtypeStruct((B, 1, N), jnp.float32),
        grid=(B, N // TN),
        in_specs=[
            pl.BlockSpec((1, G, TN), lambda b, j: (b, 0, j)),
            pl.BlockSpec((16, G + 1), lambda b, j: (0, 0)),
            pl.BlockSpec((8, 17), lambda b, j: (0, 0)),
            pl.BlockSpec((1, 9), lambda b, j: (0, 0)),
        ],
        out_specs=pl.BlockSpec((1, 1, TN), lambda b, j: (b, 0, j)),
        compiler_params=pltpu.CompilerParams(
            dimension_semantics=("parallel", "parallel"), **vmem),
    )(x1.reshape(B, G, N), p0, p1, p2)

    score2 = score.reshape(B, D * H, W)

    HWp = ((HW + 127) // 128) * 128
    pad = HWp - HW
    g5 = grid.reshape(B, nbr, H, W, 2)
    gx = g5[..., 0].reshape(B, nbr, HW)
    gy = g5[..., 1].reshape(B, nbr, HW)
    coords = jnp.concatenate([gx, gy], axis=1)
    w4 = weight.reshape(B, D, nbr, HW)
    if pad:
        coords = jnp.pad(coords, ((0, 0), (0, 0), (0, pad)))
        w4 = jnp.pad(w4, ((0, 0), (0, 0), (0, 0), (0, pad)))
    rows = ((2 * nbr + 7) // 8) * 8
    if rows != 2 * nbr:
        coords = jnp.pad(coords, ((0, 0), (0, rows - 2 * nbr), (0, 0)))

    TP = _lane_tile(HWp, 1024)
    dchunk = 8 if D % 8 == 0 else (4 if D % 4 == 0 else (2 if D % 2 == 0 else 1))
    k2 = functools.partial(_sample_agg_kernel, depth=D, height=H, width=W,
                           nbr=nbr, dchunk=dchunk)
    out = pl.pallas_call(
        k2,
        out_shape=jax.ShapeDtypeStruct((B, D, HWp), jnp.float32),
        grid=(B, HWp // TP),
        in_specs=[
            pl.BlockSpec((1, D * H, W), lambda b, p: (b, 0, 0)),
            pl.BlockSpec((1, rows, TP), lambda b, p: (b, 0, p)),
            pl.BlockSpec((1, D, nbr, TP), lambda b, p: (b, 0, 0, p)),
        ],
        out_specs=pl.BlockSpec((1, D, TP), lambda b, p: (b, 0, p)),
        compiler_params=pltpu.CompilerParams(
            dimension_semantics=("parallel", "parallel"), **vmem),
    )(score2, coords, w4)

    return out[:, :, :HW].reshape(B, D, H, W)


# speedup vs baseline: 1.0531x; 1.0531x over previous
import functools

import jax
import jax.numpy as jnp
from jax.experimental import pallas as pl
from jax.experimental.pallas import tpu as pltpu


def _mlp_kernel(x_ref, p0_ref, p1_ref, p2_ref, o_ref):
    x = x_ref[0]
    tn = x.shape[-1]
    one = jnp.ones((1, tn), jnp.float32)
    h = jnp.dot(p0_ref[...], jnp.concatenate([x, one], axis=0),
                preferred_element_type=jnp.float32)
    h = jnp.maximum(h, 0.0)
    h = jnp.dot(p1_ref[...], jnp.concatenate([h, one], axis=0),
                preferred_element_type=jnp.float32)
    h = jnp.maximum(h, 0.0)
    o_ref[0] = jnp.dot(p2_ref[...], jnp.concatenate([h, one], axis=0),
                       preferred_element_type=jnp.float32)


def _sample_agg_kernel(score_ref, coords_ref, w_ref, o_ref, *,
                       depth, height, width, nbr, dchunk):
    tp = o_ref.shape[-1]

    xs = jax.lax.broadcasted_iota(jnp.int32, (width, tp), 0).astype(jnp.float32)
    ys = jax.lax.broadcasted_iota(jnp.int32, (height, tp), 0).astype(jnp.float32)

    axs = []
    ays = []
    for n in range(nbr):
        gx = coords_ref[0, n:n + 1, :]
        gy = coords_ref[0, nbr + n:nbr + n + 1, :]

        ix = jnp.clip((gx + 1.0) * (0.5 * width) - 0.5, 0.0, width - 1.0)
        iy = jnp.clip((gy + 1.0) * (0.5 * height) - 0.5, 0.0, height - 1.0)
        x0 = jnp.floor(ix)
        y0 = jnp.floor(iy)
        wx = ix - x0
        wy = iy - y0
        x1 = jnp.minimum(x0 + 1.0, width - 1.0)
        y1 = jnp.minimum(y0 + 1.0, height - 1.0)

        axs.append(jnp.where(xs == x0, 1.0 - wx, 0.0)
                   + jnp.where(xs == x1, wx, 0.0))
        ays.append(jnp.where(ys == y0, 1.0 - wy, 0.0)
                   + jnp.where(ys == y1, wy, 0.0))

    nchunks = depth // dchunk
    accs = [None] * nchunks
    for n in range(nbr):
        ay = ays[n][None, :, :]
        for c in range(nchunks):
            rows = slice(c * dchunk * height, (c + 1) * dchunk * height)
            t = jnp.dot(score_ref[0, rows, :], axs[n],
                        preferred_element_type=jnp.float32)
            r = jnp.sum(t.reshape(dchunk, height, tp) * ay, axis=1)
            r = r * w_ref[0, c * dchunk:(c + 1) * dchunk, n, :]
            accs[c] = r if accs[c] is None else accs[c] + r
    o_ref[0] = jnp.concatenate(accs, axis=0)


def _lane_tile(n, cap):
    t = cap
    while t > 128 and n % t:
        t -= 128
    return t


def kernel(x1, grid, weight, p0, p1, p2):
    x1 = x1.astype(jnp.float32)
    grid = grid.astype(jnp.float32)
    weight = weight.astype(jnp.float32)
    p0 = p0.astype(jnp.float32)
    p1 = p1.astype(jnp.float32)
    p2 = p2.astype(jnp.float32)

    B, G, D, H, W = x1.shape
    nbr = weight.shape[2]
    HW = H * W
    N = D * HW
    vmem = dict(vmem_limit_bytes=100 * 1024 * 1024)

    TN = _lane_tile(N, 98304)
    score = pl.pallas_call(
        _mlp_kernel,
        out_shape=jax.ShapeDtypeStruct((B, 1, N), jnp.float32),
        grid=(B, N // TN),
        in_specs=[
            pl.BlockSpec((1, G, TN), lambda b, j: (b, 0, j)),
            pl.BlockSpec((16, G + 1), lambda b, j: (0, 0)),
            pl.BlockSpec((8, 17), lambda b, j: (0, 0)),
            pl.BlockSpec((1, 9), lambda b, j: (0, 0)),
        ],
        out_specs=pl.BlockSpec((1, 1, TN), lambda b, j: (b, 0, j)),
        compiler_params=pltpu.CompilerParams(
            dimension_semantics=("parallel", "parallel"), **vmem),
    )(x1.reshape(B, G, N), p0, p1, p2)

    score2 = score.reshape(B, D * H, W)

    HWp = ((HW + 127) // 128) * 128
    pad = HWp - HW
    g5 = grid.reshape(B, nbr, H, W, 2)
    gx = g5[..., 0].reshape(B, nbr, HW)
    gy = g5[..., 1].reshape(B, nbr, HW)
    coords = jnp.concatenate([gx, gy], axis=1)
    w4 = weight.reshape(B, D, nbr, HW)
    if pad:
        coords = jnp.pad(coords, ((0, 0), (0, 0), (0, pad)))
        w4 = jnp.pad(w4, ((0, 0), (0, 0), (0, 0), (0, pad)))
    rows = ((2 * nbr + 7) // 8) * 8
    if rows != 2 * nbr:
        coords = jnp.pad(coords, ((0, 0), (0, rows - 2 * nbr), (0, 0)))

    TP = _lane_tile(HWp, 512)
    dchunk = 8 if D % 8 == 0 else (4 if D % 4 == 0 else (2 if D % 2 == 0 else 1))
    k2 = functools.partial(_sample_agg_kernel, depth=D, height=H, width=W,
                           nbr=nbr, dchunk=dchunk)
    out = pl.pallas_call(
        k2,
        out_shape=jax.ShapeDtypeStruct((B, D, HWp), jnp.float32),
        grid=(B, HWp // TP),
        in_specs=[
            pl.BlockSpec((1, D * H, W), lambda b, p: (b, 0, 0)),
            pl.BlockSpec((1, rows, TP), lambda b, p: (b, 0, p)),
            pl.BlockSpec((1, D, nbr, TP), lambda b, p: (b, 0, 0, p)),
        ],
        out_specs=pl.BlockSpec((1, D, TP), lambda b, p: (b, 0, p)),
        compiler_params=pltpu.CompilerParams(
            dimension_semantics=("parallel", "parallel"), **vmem),
    )(score2, coords, w4)

    return out[:, :, :HW].reshape(B, D, H, W)


# speedup vs baseline: 1.0579x; 1.0046x over previous
import functools

import jax
import jax.numpy as jnp
from jax.experimental import pallas as pl
from jax.experimental.pallas import tpu as pltpu


def _mlp_kernel(x_ref, p0_ref, p1_ref, p2_ref, o_ref, *, g):
    x = x_ref[0]
    p0 = p0_ref[...]
    h = jnp.dot(p0[:, :g], x, preferred_element_type=jnp.float32)
    h = jnp.maximum(h + p0[:, g:g + 1], 0.0)
    p1 = p1_ref[...]
    h2 = jnp.dot(p1[:, :16], h, preferred_element_type=jnp.float32)
    h2 = jnp.maximum(h2 + p1[:, 16:17], 0.0)
    p2 = p2_ref[...]
    o_ref[0] = (jnp.dot(p2[:, :8], h2, preferred_element_type=jnp.float32)
                + p2[:, 8:9])


def _sample_agg_kernel(score_ref, coords_ref, w_ref, o_ref, *,
                       depth, height, width, nbr, dchunk):
    tp = o_ref.shape[-1]

    xs = jax.lax.broadcasted_iota(jnp.int32, (width, tp), 0).astype(jnp.float32)
    ys = jax.lax.broadcasted_iota(jnp.int32, (height, tp), 0).astype(jnp.float32)

    axs = []
    ays = []
    for n in range(nbr):
        gx = coords_ref[0, n:n + 1, :]
        gy = coords_ref[0, nbr + n:nbr + n + 1, :]

        ix = jnp.clip((gx + 1.0) * (0.5 * width) - 0.5, 0.0, width - 1.0)
        iy = jnp.clip((gy + 1.0) * (0.5 * height) - 0.5, 0.0, height - 1.0)
        x0 = jnp.floor(ix)
        y0 = jnp.floor(iy)
        wx = ix - x0
        wy = iy - y0
        x1 = jnp.minimum(x0 + 1.0, width - 1.0)
        y1 = jnp.minimum(y0 + 1.0, height - 1.0)

        axs.append(jnp.where(xs == x0, 1.0 - wx, 0.0)
                   + jnp.where(xs == x1, wx, 0.0))
        ays.append(jnp.where(ys == y0, 1.0 - wy, 0.0)
                   + jnp.where(ys == y1, wy, 0.0))

    nchunks = depth // dchunk
    accs = [None] * nchunks
    for n in range(nbr):
        ay = ays[n][None, :, :]
        for c in range(nchunks):
            rows = slice(c * dchunk * height, (c + 1) * dchunk * height)
            t = jnp.dot(score_ref[0, rows, :], axs[n],
                        preferred_element_type=jnp.float32)
            r = jnp.sum(t.reshape(dchunk, height, tp) * ay, axis=1)
            r = r * w_ref[0, c * dchunk:(c + 1) * dchunk, n, :]
            accs[c] = r if accs[c] is None else accs[c] + r
    o_ref[0] = jnp.concatenate(accs, axis=0)


def _lane_tile(n, cap):
    t = cap
    while t > 128 and n % t:
        t -= 128
    return t


def kernel(x1, grid, weight, p0, p1, p2):
    x1 = x1.astype(jnp.float32)
    grid = grid.astype(jnp.float32)
    weight = weight.astype(jnp.float32)
    p0 = p0.astype(jnp.float32)
    p1 = p1.astype(jnp.float32)
    p2 = p2.astype(jnp.float32)

    B, G, D, H, W = x1.shape
    nbr = weight.shape[2]
    HW = H * W
    N = D * HW
    vmem = dict(vmem_limit_bytes=100 * 1024 * 1024)

    TN = _lane_tile(N, 98304)
    score = pl.pallas_call(
        functools.partial(_mlp_kernel, g=G),
        out_shape=jax.ShapeDtypeStruct((B, 1, N), jnp.float32),
        grid=(B, N // TN),
        in_specs=[
            pl.BlockSpec((1, G, TN), lambda b, j: (b, 0, j)),
            pl.BlockSpec((16, G + 1), lambda b, j: (0, 0)),
            pl.BlockSpec((8, 17), lambda b, j: (0, 0)),
            pl.BlockSpec((1, 9), lambda b, j: (0, 0)),
        ],
        out_specs=pl.BlockSpec((1, 1, TN), lambda b, j: (b, 0, j)),
        compiler_params=pltpu.CompilerParams(
            dimension_semantics=("parallel", "parallel"), **vmem),
    )(x1.reshape(B, G, N), p0, p1, p2)

    score2 = score.reshape(B, D * H, W)

    HWp = ((HW + 127) // 128) * 128
    pad = HWp - HW
    g5 = grid.reshape(B, nbr, H, W, 2)
    gx = g5[..., 0].reshape(B, nbr, HW)
    gy = g5[..., 1].reshape(B, nbr, HW)
    coords = jnp.concatenate([gx, gy], axis=1)
    w4 = weight.reshape(B, D, nbr, HW)
    if pad:
        coords = jnp.pad(coords, ((0, 0), (0, 0), (0, pad)))
        w4 = jnp.pad(w4, ((0, 0), (0, 0), (0, 0), (0, pad)))
    rows = ((2 * nbr + 7) // 8) * 8
    if rows != 2 * nbr:
        coords = jnp.pad(coords, ((0, 0), (0, rows - 2 * nbr), (0, 0)))

    TP = _lane_tile(HWp, 512)
    dchunk = 8 if D % 8 == 0 else (4 if D % 4 == 0 else (2 if D % 2 == 0 else 1))
    k2 = functools.partial(_sample_agg_kernel, depth=D, height=H, width=W,
                           nbr=nbr, dchunk=dchunk)
    out = pl.pallas_call(
        k2,
        out_shape=jax.ShapeDtypeStruct((B, D, HWp), jnp.float32),
        grid=(B, HWp // TP),
        in_specs=[
            pl.BlockSpec((1, D * H, W), lambda b, p: (b, 0, 0)),
            pl.BlockSpec((1, rows, TP), lambda b, p: (b, 0, p)),
            pl.BlockSpec((1, D, nbr, TP), lambda b, p: (b, 0, 0, p)),
        ],
        out_specs=pl.BlockSpec((1, D, TP), lambda b, p: (b, 0, p)),
        compiler_params=pltpu.CompilerParams(
            dimension_semantics=("parallel", "parallel"), **vmem),
    )(score2, coords, w4)

    return out[:, :, :HW].reshape(B, D, H, W)


# speedup vs baseline: 1.0582x; 1.0003x over previous
import functools

import jax
import jax.numpy as jnp
from jax.experimental import pallas as pl
from jax.experimental.pallas import tpu as pltpu


def _mlp_kernel(x_ref, p0_ref, p1_ref, p2_ref, o_ref, *, g):
    x = x_ref[0]
    p0 = p0_ref[...]
    h = jnp.dot(p0[:, :g], x, preferred_element_type=jnp.float32)
    h = jnp.maximum(h + p0[:, g:g + 1], 0.0)
    p1 = p1_ref[...]
    h2 = jnp.dot(p1[:, :16], h, preferred_element_type=jnp.float32)
    h2 = jnp.maximum(h2 + p1[:, 16:17], 0.0)
    p2 = p2_ref[...]
    o_ref[0] = (jnp.dot(p2[:, :8], h2, preferred_element_type=jnp.float32)
                + p2[:, 8:9])


def _sample_agg_kernel(score_ref, coords_ref, w_ref, o_ref, *,
                       depth, height, width, nbr, dchunk):
    tp = o_ref.shape[-1]

    xs = jax.lax.broadcasted_iota(jnp.int32, (width, tp), 0).astype(jnp.float32)
    ys = jax.lax.broadcasted_iota(jnp.int32, (height, tp), 0).astype(jnp.float32)

    axs = []
    ays = []
    for n in range(nbr):
        gx = coords_ref[0, n:n + 1, :]
        gy = coords_ref[0, nbr + n:nbr + n + 1, :]

        ix = jnp.clip((gx + 1.0) * (0.5 * width) - 0.5, 0.0, width - 1.0)
        iy = jnp.clip((gy + 1.0) * (0.5 * height) - 0.5, 0.0, height - 1.0)
        x0 = jnp.floor(ix)
        y0 = jnp.floor(iy)
        wx = ix - x0
        wy = iy - y0
        x1 = jnp.minimum(x0 + 1.0, width - 1.0)
        y1 = jnp.minimum(y0 + 1.0, height - 1.0)

        axs.append(jnp.where(xs == x0, 1.0 - wx, 0.0)
                   + jnp.where(xs == x1, wx, 0.0))
        ays.append(jnp.where(ys == y0, 1.0 - wy, 0.0)
                   + jnp.where(ys == y1, wy, 0.0))

    nchunks = depth // dchunk
    accs = [None] * nchunks
    for n in range(nbr):
        ay = ays[n][None, :, :]
        for c in range(nchunks):
            rows = slice(c * dchunk * height, (c + 1) * dchunk * height)
            t = jnp.dot(score_ref[0, rows, :], axs[n],
                        preferred_element_type=jnp.float32)
            r = jnp.sum(t.reshape(dchunk, height, tp) * ay, axis=1)
            r = r * w_ref[0, c * dchunk:(c + 1) * dchunk, n, :]
            accs[c] = r if accs[c] is None else accs[c] + r
    o_ref[0] = jnp.concatenate(accs, axis=0)


def _lane_tile(n, cap):
    t = cap
    while t > 128 and n % t:
        t -= 128
    return t


def kernel(x1, grid, weight, p0, p1, p2):
    x1 = x1.astype(jnp.float32)
    grid = grid.astype(jnp.float32)
    weight = weight.astype(jnp.float32)
    p0 = p0.astype(jnp.float32)
    p1 = p1.astype(jnp.float32)
    p2 = p2.astype(jnp.float32)

    B, G, D, H, W = x1.shape
    nbr = weight.shape[2]
    HW = H * W
    N = D * HW
    vmem = dict(vmem_limit_bytes=100 * 1024 * 1024)

    TN = _lane_tile(N, 98304)
    score = pl.pallas_call(
        functools.partial(_mlp_kernel, g=G),
        out_shape=jax.ShapeDtypeStruct((B, 1, N), jnp.float32),
        grid=(B, N // TN),
        in_specs=[
            pl.BlockSpec((1, G, TN), lambda b, j: (b, 0, j)),
            pl.BlockSpec((16, G + 1), lambda b, j: (0, 0)),
            pl.BlockSpec((8, 17), lambda b, j: (0, 0)),
            pl.BlockSpec((1, 9), lambda b, j: (0, 0)),
        ],
        out_specs=pl.BlockSpec((1, 1, TN), lambda b, j: (b, 0, j)),
        compiler_params=pltpu.CompilerParams(
            dimension_semantics=("parallel", "parallel"), **vmem),
    )(x1.reshape(B, G, N), p0, p1, p2)

    score2 = score.reshape(B, D * H, W)

    HWp = ((HW + 127) // 128) * 128
    pad = HWp - HW
    coords = jnp.transpose(grid.reshape(B, nbr, HW, 2),
                           (0, 3, 1, 2)).reshape(B, 2 * nbr, HW)
    w4 = weight.reshape(B, D, nbr, HW)
    if pad:
        coords = jnp.pad(coords, ((0, 0), (0, 0), (0, pad)))
        w4 = jnp.pad(w4, ((0, 0), (0, 0), (0, 0), (0, pad)))
    rows = ((2 * nbr + 7) // 8) * 8
    if rows != 2 * nbr:
        coords = jnp.pad(coords, ((0, 0), (0, rows - 2 * nbr), (0, 0)))

    TP = _lane_tile(HWp, 512)
    dchunk = 8 if D % 8 == 0 else (4 if D % 4 == 0 else (2 if D % 2 == 0 else 1))
    k2 = functools.partial(_sample_agg_kernel, depth=D, height=H, width=W,
                           nbr=nbr, dchunk=dchunk)
    out = pl.pallas_call(
        k2,
        out_shape=jax.ShapeDtypeStruct((B, D, HWp), jnp.float32),
        grid=(B, HWp // TP),
        in_specs=[
            pl.BlockSpec((1, D * H, W), lambda b, p: (b, 0, 0)),
            pl.BlockSpec((1, rows, TP), lambda b, p: (b, 0, p)),
            pl.BlockSpec((1, D, nbr, TP), lambda b, p: (b, 0, 0, p)),
        ],
        out_specs=pl.BlockSpec((1, D, TP), lambda b, p: (b, 0, p)),
        compiler_params=pltpu.CompilerParams(
            dimension_semantics=("parallel", "parallel"), **vmem),
    )(score2, coords, w4)

    return out[:, :, :HW].reshape(B, D, H, W)
